# baseline (speedup 1.0000x reference)
import functools

import jax
import jax.numpy as jnp
from jax import lax
from jax.experimental import pallas as pl
from jax.experimental.pallas import tpu as pltpu
from jax.experimental.pallas import tpu_sc as plsc

B = 16384
D = 32
H0 = 64
H1 = 32
NW = 32
BPW = B // NW
CH = 128
NCH = BPW // CH


def _gather_sc(user, item, user_table, item_table):
    mesh = plsc.VectorSubcoreMesh(core_axis_name="c", subcore_axis_name="s")

    @functools.partial(
        pl.kernel,
        mesh=mesh,
        out_type=(
            jax.ShapeDtypeStruct((B, D), jnp.float32),
            jax.ShapeDtypeStruct((B, D), jnp.float32),
        ),
        scratch_types=[
            pltpu.VMEM((NCH, CH), jnp.int32),
            pltpu.VMEM((NCH, CH), jnp.int32),
            pltpu.VMEM((BPW, D), jnp.float32),
            pltpu.VMEM((BPW, D), jnp.float32),
            pltpu.SemaphoreType.DMA,
        ],
        compiler_params=pltpu.CompilerParams(use_tc_tiling_on_sc=False),
    )
    def gather_kernel(user_hbm, item_hbm, ut_hbm, it_hbm, ue_hbm, ie_hbm,
                      uidx, iidx, urows, irows, sem):
        wid = lax.axis_index("s") * 2 + lax.axis_index("c")
        base = wid * BPW
        for j in range(NCH):
            pltpu.sync_copy(user_hbm.at[pl.ds(base + j * CH, CH)], uidx.at[j])
            pltpu.sync_copy(item_hbm.at[pl.ds(base + j * CH, CH)], iidx.at[j])
        copies = []
        for j in range(NCH):
            copies.append(pltpu.async_copy(
                ut_hbm.at[uidx.at[j]], urows.at[pl.ds(j * CH, CH)], sem))
            copies.append(pltpu.async_copy(
                it_hbm.at[iidx.at[j]], irows.at[pl.ds(j * CH, CH)], sem))
        for c in copies:
            c.wait()
        pltpu.sync_copy(urows, ue_hbm.at[pl.ds(base, BPW)])
        pltpu.sync_copy(irows, ie_hbm.at[pl.ds(base, BPW)])

    return gather_kernel(user, item, user_table, item_table)


def _mlp_body(ue_ref, ie_ref, w0u_ref, w0i_ref, b0_ref, w1_ref, b1_ref,
              wo_ref, bo_ref, out_ref):
    x0 = jnp.dot(ue_ref[...], w0u_ref[...], preferred_element_type=jnp.float32)
    x0 += jnp.dot(ie_ref[...], w0i_ref[...], preferred_element_type=jnp.float32)
    x0 = jnp.maximum(x0 + b0_ref[...], 0.0)
    x1 = jnp.maximum(
        jnp.dot(x0, w1_ref[...], preferred_element_type=jnp.float32)
        + b1_ref[...], 0.0)
    z = jnp.sum(x1 * wo_ref[...], axis=1, keepdims=True) + bo_ref[...]
    out_ref[...] = 1.0 / (1.0 + jnp.exp(-z))


def _mlp_tc(ue, ie, W0u, W0i, b0, W1, b1, wout_row, bout, interpret=False):
    Bb = 2048
    return pl.pallas_call(
        _mlp_body,
        grid=(B // Bb,),
        in_specs=[
            pl.BlockSpec((Bb, D), lambda i: (i, 0)),
            pl.BlockSpec((Bb, D), lambda i: (i, 0)),
            pl.BlockSpec((D, H0), lambda i: (0, 0)),
            pl.BlockSpec((D, H0), lambda i: (0, 0)),
            pl.BlockSpec((1, H0), lambda i: (0, 0)),
            pl.BlockSpec((H0, H1), lambda i: (0, 0)),
            pl.BlockSpec((1, H1), lambda i: (0, 0)),
            pl.BlockSpec((1, H1), lambda i: (0, 0)),
            pl.BlockSpec((1, 1), lambda i: (0, 0)),
        ],
        out_specs=pl.BlockSpec((Bb, 1), lambda i: (i, 0)),
        out_shape=jax.ShapeDtypeStruct((B, 1), jnp.float32),
        interpret=interpret,
    )(ue, ie, W0u, W0i, b0, W1, b1, wout_row, bout)


def kernel(user, item, user_table, item_table, W0, b0, W1, b1, Wout, bout):
    user = user.astype(jnp.int32)
    item = item.astype(jnp.int32)
    ue, ie = _gather_sc(user, item, user_table, item_table)
    return _mlp_tc(ue, ie, W0[:D], W0[D:], b0.reshape(1, H0), W1,
                   b1.reshape(1, H1), Wout.reshape(1, H1),
                   bout.reshape(1, 1))
